# Initial kernel scaffold; baseline (speedup 1.0000x reference)
import functools

import jax
import jax.numpy as jnp
import numpy as np
from jax import lax
from jax.experimental import pallas as pl
from jax.experimental.pallas import tpu as pltpu
from jax.experimental.pallas import tpu_sc as plsc

_D = 64
_LANES = 16
_NW = 32
_C = 512
_SUB = 128


def _rsqrt_nr(x):
    i = plsc.bitcast(x, jnp.int32)
    y = plsc.bitcast(jnp.int32(0x5F3759DF) - (i >> 1), jnp.float32)
    for _ in range(3):
        y = y * (1.5 - 0.5 * x * y * y)
    return y


def _sc_body(n_per_w, n_chunks):
    def body(idx_hbm, table_hbm, out_hbm, idx1_v, idx2_v, rows1_v, rows2_v,
             out_v, sem):
        wid = lax.axis_index("s") * 2 + lax.axis_index("c")

        def chunk(k, carry):
            base = wid * n_per_w + k * _C
            b128 = base // _SUB
            pltpu.sync_copy(idx_hbm.at[0, pl.ds(b128, _C // _SUB), :], idx1_v)
            pltpu.sync_copy(idx_hbm.at[1, pl.ds(b128, _C // _SUB), :], idx2_v)
            copies = []
            for j in range(_C // _SUB):
                copies.append(pltpu.async_copy(
                    table_hbm.at[idx1_v.at[j]],
                    rows1_v.at[pl.ds(j * _SUB, _SUB)], sem))
                copies.append(pltpu.async_copy(
                    table_hbm.at[idx2_v.at[j]],
                    rows2_v.at[pl.ds(j * _SUB, _SUB)], sem))
            for cp in copies:
                cp.wait()

            def group(g, carry2):
                p0 = g * _LANES
                d12 = jnp.zeros((_LANES,), jnp.float32)
                d11 = jnp.zeros((_LANES,), jnp.float32)
                d22 = jnp.zeros((_LANES,), jnp.float32)
                for j in range(_LANES):
                    p = p0 + j
                    a = [rows1_v[p, pl.ds(t * _LANES, _LANES)]
                         for t in range(_D // _LANES)]
                    b = [rows2_v[p, pl.ds(t * _LANES, _LANES)]
                         for t in range(_D // _LANES)]
                    s12 = a[0] * b[0] + a[1] * b[1] + a[2] * b[2] + a[3] * b[3]
                    s11 = a[0] * a[0] + a[1] * a[1] + a[2] * a[2] + a[3] * a[3]
                    s22 = b[0] * b[0] + b[1] * b[1] + b[2] * b[2] + b[3] * b[3]
                    m = np.arange(_LANES) == j
                    d12 = jnp.where(m, jnp.sum(s12), d12)
                    d11 = jnp.where(m, jnp.sum(s11), d11)
                    d22 = jnp.where(m, jnp.sum(s22), d22)
                prod = jnp.maximum(d11, 1e-24) * jnp.maximum(d22, 1e-24)
                out_v[pl.ds(p0, _LANES)] = d12 * _rsqrt_nr(prod)
                return carry2

            lax.fori_loop(0, _C // _LANES, group, 0)
            pltpu.sync_copy(out_v, out_hbm.at[pl.ds(base, _C)])
            return carry

        lax.fori_loop(0, n_chunks, chunk, 0)

    return body


@jax.jit
def kernel(inputs, embeddings):
    two, B, L = inputs.shape
    V, D = embeddings.shape
    N = B * L
    n_per_w = N // _NW
    n_chunks = n_per_w // _C

    idx3 = inputs.reshape(2, N // _SUB, _SUB)

    mesh = plsc.VectorSubcoreMesh(core_axis_name="c", subcore_axis_name="s")
    run = pl.kernel(
        _sc_body(n_per_w, n_chunks),
        out_type=jax.ShapeDtypeStruct((N,), jnp.float32),
        mesh=mesh,
        scratch_types=[
            pltpu.VMEM((_C // _SUB, _SUB), jnp.int32),
            pltpu.VMEM((_C // _SUB, _SUB), jnp.int32),
            pltpu.VMEM((_C, _D), jnp.float32),
            pltpu.VMEM((_C, _D), jnp.float32),
            pltpu.VMEM((_C,), jnp.float32),
            pltpu.SemaphoreType.DMA,
        ],
    )
    out = run(idx3, embeddings)
    return out.reshape(B, L)


# speedup vs baseline: 1.5590x; 1.5590x over previous
import functools

import jax
import jax.numpy as jnp
import numpy as np
from jax import lax
from jax.experimental import pallas as pl
from jax.experimental.pallas import tpu as pltpu
from jax.experimental.pallas import tpu_sc as plsc

_D = 64
_LANES = 16
_NW = 32
_C = 512
_SUB = 128


_GDN = lax.GatherDimensionNumbers(
    offset_dims=(), collapsed_slice_dims=(0,), start_index_map=(0,))


def _shuffle(x, idx):
    return lax.gather(x, idx[:, None], dimension_numbers=_GDN,
                      slice_sizes=(1,),
                      mode=lax.GatherScatterMode.PROMISE_IN_BOUNDS)


def _hsum(x, lanes):
    for k in (8, 4, 2, 1):
        x = x + _shuffle(x, lanes ^ k)
    return x


def _rsqrt_nr(x):
    i = lax.bitcast_convert_type(x, jnp.int32)
    y = lax.bitcast_convert_type(jnp.int32(0x5F3759DF) - (i >> 1),
                                 jnp.float32)
    for _ in range(3):
        y = y * (1.5 - 0.5 * x * y * y)
    return y


def _sc_body(n, n_per_w, n_chunks):
    def body(idx_hbm, table_hbm, out_hbm, idx1_v, idx2_v, rows1_v, rows2_v,
             out_v, sem):
        wid = lax.axis_index("s") * 2 + lax.axis_index("c")

        def chunk(k, carry):
            base = wid * n_per_w + k * _C
            pltpu.sync_copy(idx_hbm.at[pl.ds(base, _C)], idx1_v)
            pltpu.sync_copy(idx_hbm.at[pl.ds(n + base, _C)], idx2_v)
            copies = []
            for j in range(_C // _SUB):
                copies.append(pltpu.async_copy(
                    table_hbm.at[idx1_v.at[pl.ds(j * _SUB, _SUB)]],
                    rows1_v.at[pl.ds(j * _SUB, _SUB)], sem))
                copies.append(pltpu.async_copy(
                    table_hbm.at[idx2_v.at[pl.ds(j * _SUB, _SUB)]],
                    rows2_v.at[pl.ds(j * _SUB, _SUB)], sem))
            for cp in copies:
                cp.wait()

            def group(g, carry2):
                lanes = lax.iota(jnp.int32, _LANES)
                p0 = g * _LANES
                d12 = jnp.zeros((_LANES,), jnp.float32)
                d11 = jnp.zeros((_LANES,), jnp.float32)
                d22 = jnp.zeros((_LANES,), jnp.float32)
                for j in range(_LANES):
                    p = p0 + j
                    a = [rows1_v[p, pl.ds(t * _LANES, _LANES)]
                         for t in range(_D // _LANES)]
                    b = [rows2_v[p, pl.ds(t * _LANES, _LANES)]
                         for t in range(_D // _LANES)]
                    s12 = a[0] * b[0] + a[1] * b[1] + a[2] * b[2] + a[3] * b[3]
                    s11 = a[0] * a[0] + a[1] * a[1] + a[2] * a[2] + a[3] * a[3]
                    s22 = b[0] * b[0] + b[1] * b[1] + b[2] * b[2] + b[3] * b[3]
                    m = lanes == j
                    d12 = jnp.where(m, _hsum(s12, lanes), d12)
                    d11 = jnp.where(m, _hsum(s11, lanes), d11)
                    d22 = jnp.where(m, _hsum(s22, lanes), d22)
                prod = jnp.maximum(d11, 1e-24) * jnp.maximum(d22, 1e-24)
                out_v[pl.ds(p0, _LANES)] = d12 * _rsqrt_nr(prod)
                return carry2

            lax.fori_loop(0, _C // _LANES, group, 0)
            pltpu.sync_copy(out_v, out_hbm.at[pl.ds(base, _C)])
            return carry

        lax.fori_loop(0, n_chunks, chunk, 0)

    return body


@jax.jit
def kernel(inputs, embeddings):
    two, B, L = inputs.shape
    V, D = embeddings.shape
    N = B * L
    n_per_w = N // _NW
    n_chunks = n_per_w // _C

    idx_flat = inputs.reshape(2 * N)

    mesh = plsc.VectorSubcoreMesh(core_axis_name="c", subcore_axis_name="s")
    run = pl.kernel(
        _sc_body(N, n_per_w, n_chunks),
        out_type=jax.ShapeDtypeStruct((N,), jnp.float32),
        mesh=mesh,
        compiler_params=pltpu.CompilerParams(use_tc_tiling_on_sc=False),
        scratch_types=[
            pltpu.VMEM((_C,), jnp.int32),
            pltpu.VMEM((_C,), jnp.int32),
            pltpu.VMEM((_C, _D), jnp.float32),
            pltpu.VMEM((_C, _D), jnp.float32),
            pltpu.VMEM((_C,), jnp.float32),
            pltpu.SemaphoreType.DMA,
        ],
    )
    out = run(idx_flat, embeddings)
    return out.reshape(B, L)


# speedup vs baseline: 1.8207x; 1.1679x over previous
import functools

import jax
import jax.numpy as jnp
import numpy as np
from jax import lax
from jax.experimental import pallas as pl
from jax.experimental.pallas import tpu as pltpu
from jax.experimental.pallas import tpu_sc as plsc

_D = 64
_LANES = 16
_NW = 32
_C = 256
_SUB = 128
_BITREV = (0, 8, 4, 12, 2, 10, 6, 14, 1, 9, 5, 13, 3, 11, 7, 15)

_GDN = lax.GatherDimensionNumbers(
    offset_dims=(), collapsed_slice_dims=(0,), start_index_map=(0,))


def _shuffle(x, idx):
    return lax.gather(x, idx[:, None], dimension_numbers=_GDN,
                      slice_sizes=(1,),
                      mode=lax.GatherScatterMode.PROMISE_IN_BOUNDS)


def _rsqrt_nr(x):
    i = lax.bitcast_convert_type(x, jnp.int32)
    y = lax.bitcast_convert_type(jnp.int32(0x5F3759DF) - (i >> 1),
                                 jnp.float32)
    for _ in range(3):
        y = y * (1.5 - 0.5 * x * y * y)
    return y


def _reduce16(vecs, lanes):
    level = 0
    for k in (8, 4, 2, 1):
        mask = (lanes & k) == 0
        perm = lanes ^ k
        nxt = []
        for j in range(len(vecs) // 2):
            a, b = vecs[2 * j], vecs[2 * j + 1]
            nxt.append(jnp.where(mask, a + _shuffle(a, perm),
                                 b + _shuffle(b, perm)))
        vecs = nxt
    return vecs[0]


def _sc_body(n, n_per_w, n_chunks):
    nsub = _C // _SUB

    def body(idx_hbm, table_hbm, out_hbm, idx1_v, idx2_v, r1a, r2a, r1b,
             r2b, oa, ob, sem_a, sem_b, sem_o):
        wid = lax.axis_index("s") * 2 + lax.axis_index("c")
        base_w = wid * n_per_w
        pltpu.sync_copy(idx_hbm.at[pl.ds(base_w, n_per_w)], idx1_v)
        pltpu.sync_copy(idx_hbm.at[pl.ds(n + base_w, n_per_w)], idx2_v)

        def fire(k, r1, r2, sem):
            for j in range(nsub):
                off = k * _C + j * _SUB
                pltpu.async_copy(
                    table_hbm.at[idx1_v.at[pl.ds(off, _SUB)]],
                    r1.at[pl.ds(j * _SUB, _SUB)], sem)
                pltpu.async_copy(
                    table_hbm.at[idx2_v.at[pl.ds(off, _SUB)]],
                    r2.at[pl.ds(j * _SUB, _SUB)], sem)

        def drain_rows(r1, r2, sem):
            for j in range(nsub):
                pltpu.make_async_copy(
                    table_hbm.at[pl.ds(0, _SUB)],
                    r1.at[pl.ds(j * _SUB, _SUB)], sem).wait()
                pltpu.make_async_copy(
                    table_hbm.at[pl.ds(0, _SUB)],
                    r2.at[pl.ds(j * _SUB, _SUB)], sem).wait()

        def drain_out(ov):
            pltpu.make_async_copy(
                out_hbm.at[pl.ds(0, _C)], ov, sem_o).wait()

        def compute(k, r1, r2, ov):
            def group(g, carry):
                lanes = lax.iota(jnp.int32, _LANES)
                p0 = g * _LANES
                s12s, s11s, s22s = [], [], []
                for j in _BITREV:
                    p = p0 + j
                    a = [r1[p, pl.ds(t * _LANES, _LANES)]
                         for t in range(_D // _LANES)]
                    b = [r2[p, pl.ds(t * _LANES, _LANES)]
                         for t in range(_D // _LANES)]
                    s12s.append((a[0] * b[0] + a[1] * b[1])
                                + (a[2] * b[2] + a[3] * b[3]))
                    s11s.append((a[0] * a[0] + a[1] * a[1])
                                + (a[2] * a[2] + a[3] * a[3]))
                    s22s.append((b[0] * b[0] + b[1] * b[1])
                                + (b[2] * b[2] + b[3] * b[3]))
                d12 = _reduce16(s12s, lanes)
                d11 = _reduce16(s11s, lanes)
                d22 = _reduce16(s22s, lanes)
                prod = jnp.maximum(d11, 1e-24) * jnp.maximum(d22, 1e-24)
                ov[pl.ds(p0, _LANES)] = d12 * _rsqrt_nr(prod)
                return carry

            lax.fori_loop(0, _C // _LANES, group, 0)
            pltpu.async_copy(ov, out_hbm.at[pl.ds(base_w + k * _C, _C)],
                             sem_o)

        fire(0, r1a, r2a, sem_a)
        fire(1, r1b, r2b, sem_b)

        def step(i, carry):
            k0 = 2 * i
            k1 = k0 + 1
            drain_rows(r1a, r2a, sem_a)

            @pl.when(i > 0)
            def _():
                drain_out(oa)

            compute(k0, r1a, r2a, oa)

            @pl.when(k0 + 2 < n_chunks)
            def _():
                fire(k0 + 2, r1a, r2a, sem_a)

            drain_rows(r1b, r2b, sem_b)

            @pl.when(i > 0)
            def _():
                drain_out(ob)

            compute(k1, r1b, r2b, ob)

            @pl.when(k1 + 2 < n_chunks)
            def _():
                fire(k1 + 2, r1b, r2b, sem_b)

            return carry

        lax.fori_loop(0, n_chunks // 2, step, 0)
        drain_out(oa)
        drain_out(ob)

    return body


@jax.jit
def kernel(inputs, embeddings):
    two, B, L = inputs.shape
    V, D = embeddings.shape
    N = B * L
    n_per_w = N // _NW
    n_chunks = n_per_w // _C

    idx_flat = inputs.reshape(2 * N)

    mesh = plsc.VectorSubcoreMesh(core_axis_name="c", subcore_axis_name="s")
    run = pl.kernel(
        _sc_body(N, n_per_w, n_chunks),
        out_type=jax.ShapeDtypeStruct((N,), jnp.float32),
        mesh=mesh,
        compiler_params=pltpu.CompilerParams(use_tc_tiling_on_sc=False),
        scratch_types=[
            pltpu.VMEM((n_per_w,), jnp.int32),
            pltpu.VMEM((n_per_w,), jnp.int32),
            pltpu.VMEM((_C, _D), jnp.float32),
            pltpu.VMEM((_C, _D), jnp.float32),
            pltpu.VMEM((_C, _D), jnp.float32),
            pltpu.VMEM((_C, _D), jnp.float32),
            pltpu.VMEM((_C,), jnp.float32),
            pltpu.VMEM((_C,), jnp.float32),
            pltpu.SemaphoreType.DMA,
            pltpu.SemaphoreType.DMA,
            pltpu.SemaphoreType.DMA,
        ],
    )
    out = run(idx_flat, embeddings)
    return out.reshape(B, L)


# speedup vs baseline: 2.2029x; 1.2099x over previous
import functools

import jax
import jax.numpy as jnp
import numpy as np
from jax import lax
from jax.experimental import pallas as pl
from jax.experimental.pallas import tpu as pltpu
from jax.experimental.pallas import tpu_sc as plsc

_D = 64
_LANES = 16
_NW = 32
_C = 128
_SUB = 128
_VB = 8192
_BITREV = (0, 8, 4, 12, 2, 10, 6, 14, 1, 9, 5, 13, 3, 11, 7, 15)

_GDN = lax.GatherDimensionNumbers(
    offset_dims=(), collapsed_slice_dims=(0,), start_index_map=(0,))


def _shuffle(x, idx):
    return lax.gather(x, idx[:, None], dimension_numbers=_GDN,
                      slice_sizes=(1,),
                      mode=lax.GatherScatterMode.PROMISE_IN_BOUNDS)


def _rsqrt_nr(x):
    i = lax.bitcast_convert_type(x, jnp.int32)
    y = lax.bitcast_convert_type(jnp.int32(0x5F3759DF) - (i >> 1),
                                 jnp.float32)
    for _ in range(3):
        y = y * (1.5 - 0.5 * x * y * y)
    return y


def _reduce16(vecs, lanes):
    for k in (8, 4, 2, 1):
        mask = (lanes & k) == 0
        perm = lanes ^ k
        vecs = [jnp.where(mask, a + _shuffle(a, perm),
                          b + _shuffle(b, perm))
                for a, b in zip(vecs[0::2], vecs[1::2])]
    return vecs[0]


def _tc_relayout(emb_t, V, D):
    def body(src_ref, dst_ref):
        t = src_ref[...].T.reshape(_VB // 2, 2, D)
        dst_ref[...] = jnp.concatenate([t[:, 0, :], t[:, 1, :]], axis=1)

    grid = (V + _VB - 1) // _VB
    return pl.pallas_call(
        body,
        grid=(grid,),
        in_specs=[pl.BlockSpec((D, _VB), lambda i: (0, i))],
        out_specs=pl.BlockSpec((_VB // 2, 2 * D), lambda i: (i, 0)),
        out_shape=jax.ShapeDtypeStruct((V // 2, 2 * D), jnp.float32),
    )(emb_t)


def _sc_body(n, n_per_w, n_chunks):
    def body(idx_hbm, table_hbm, out_hbm, idx1_v, idx2_v, vdx1_v, vdx2_v,
             r1a, r2a, r1b, r2b, oa, ob, sem_a, sem_b, sem_o):
        wid = lax.axis_index("s") * 2 + lax.axis_index("c")
        base_w = wid * n_per_w
        pltpu.sync_copy(idx_hbm.at[pl.ds(base_w, n_per_w)], idx1_v)
        pltpu.sync_copy(idx_hbm.at[pl.ds(n + base_w, n_per_w)], idx2_v)

        def halve(i, carry):
            o = i * _LANES
            vdx1_v[pl.ds(o, _LANES)] = idx1_v[pl.ds(o, _LANES)] >> 1
            vdx2_v[pl.ds(o, _LANES)] = idx2_v[pl.ds(o, _LANES)] >> 1
            return carry

        lax.fori_loop(0, n_per_w // _LANES, halve, 0)

        def fire(k, r1, r2, sem):
            off = k * _C
            pltpu.async_copy(
                table_hbm.at[vdx1_v.at[pl.ds(off, _SUB)]], r1, sem)
            pltpu.async_copy(
                table_hbm.at[vdx2_v.at[pl.ds(off, _SUB)]], r2, sem)

        def drain_rows(r1, r2, sem):
            pltpu.make_async_copy(
                table_hbm.at[pl.ds(0, _SUB)], r1, sem).wait()
            pltpu.make_async_copy(
                table_hbm.at[pl.ds(0, _SUB)], r2, sem).wait()

        def drain_out(ov):
            pltpu.make_async_copy(
                out_hbm.at[pl.ds(0, _C)], ov, sem_o).wait()

        def compute(k, r1, r2, ov):
            def group(g, carry):
                lanes = lax.iota(jnp.int32, _LANES)
                p0 = g * _LANES
                h1v = (idx1_v[pl.ds(k * _C + p0, _LANES)] & 1) * _D
                h2v = (idx2_v[pl.ds(k * _C + p0, _LANES)] & 1) * _D
                s12s, s11s, s22s = [], [], []
                for j in _BITREV:
                    p = p0 + j
                    h1 = h1v[j]
                    h2 = h2v[j]
                    a = [r1[p, pl.ds(h1 + t * _LANES, _LANES)]
                         for t in range(_D // _LANES)]
                    b = [r2[p, pl.ds(h2 + t * _LANES, _LANES)]
                         for t in range(_D // _LANES)]
                    s12s.append((a[0] * b[0] + a[1] * b[1])
                                + (a[2] * b[2] + a[3] * b[3]))
                    s11s.append((a[0] * a[0] + a[1] * a[1])
                                + (a[2] * a[2] + a[3] * a[3]))
                    s22s.append((b[0] * b[0] + b[1] * b[1])
                                + (b[2] * b[2] + b[3] * b[3]))
                d12 = _reduce16(s12s, lanes)
                d11 = _reduce16(s11s, lanes)
                d22 = _reduce16(s22s, lanes)
                prod = jnp.maximum(d11, 1e-24) * jnp.maximum(d22, 1e-24)
                ov[pl.ds(p0, _LANES)] = d12 * _rsqrt_nr(prod)
                return carry

            lax.fori_loop(0, _C // _LANES, group, 0)
            pltpu.async_copy(ov, out_hbm.at[pl.ds(base_w + k * _C, _C)],
                             sem_o)

        fire(0, r1a, r2a, sem_a)
        fire(1, r1b, r2b, sem_b)

        def step(i, carry):
            k0 = 2 * i
            k1 = k0 + 1
            drain_rows(r1a, r2a, sem_a)

            @pl.when(i > 0)
            def _():
                drain_out(oa)

            compute(k0, r1a, r2a, oa)

            @pl.when(k0 + 2 < n_chunks)
            def _():
                fire(k0 + 2, r1a, r2a, sem_a)

            drain_rows(r1b, r2b, sem_b)

            @pl.when(i > 0)
            def _():
                drain_out(ob)

            compute(k1, r1b, r2b, ob)

            @pl.when(k1 + 2 < n_chunks)
            def _():
                fire(k1 + 2, r1b, r2b, sem_b)

            return carry

        lax.fori_loop(0, n_chunks // 2, step, 0)
        drain_out(oa)
        drain_out(ob)

    return body


@jax.jit
def kernel(inputs, embeddings):
    two, B, L = inputs.shape
    V, D = embeddings.shape
    N = B * L
    n_per_w = N // _NW
    n_chunks = n_per_w // _C

    idx_flat = inputs.reshape(2 * N)
    table2 = _tc_relayout(embeddings.T, V, D)

    mesh = plsc.VectorSubcoreMesh(core_axis_name="c", subcore_axis_name="s")
    run = pl.kernel(
        _sc_body(N, n_per_w, n_chunks),
        out_type=jax.ShapeDtypeStruct((N,), jnp.float32),
        mesh=mesh,
        compiler_params=pltpu.CompilerParams(use_tc_tiling_on_sc=True),
        scratch_types=[
            pltpu.VMEM((n_per_w,), jnp.int32),
            pltpu.VMEM((n_per_w,), jnp.int32),
            pltpu.VMEM((n_per_w,), jnp.int32),
            pltpu.VMEM((n_per_w,), jnp.int32),
            pltpu.VMEM((_C, 2 * _D), jnp.float32),
            pltpu.VMEM((_C, 2 * _D), jnp.float32),
            pltpu.VMEM((_C, 2 * _D), jnp.float32),
            pltpu.VMEM((_C, 2 * _D), jnp.float32),
            pltpu.VMEM((_C,), jnp.float32),
            pltpu.VMEM((_C,), jnp.float32),
            pltpu.SemaphoreType.DMA,
            pltpu.SemaphoreType.DMA,
            pltpu.SemaphoreType.DMA,
        ],
    )
    out = run(idx_flat, table2)
    return out.reshape(B, L)


# speedup vs baseline: 3.1255x; 1.4188x over previous
import functools

import jax
import jax.numpy as jnp
import numpy as np
from jax import lax
from jax.experimental import pallas as pl
from jax.experimental.pallas import tpu as pltpu
from jax.experimental.pallas import tpu_sc as plsc

_D = 64
_LANES = 16
_NW = 32
_C = 128
_SUB = 128
_VB = 8192
_BITREV = (0, 8, 4, 12, 2, 10, 6, 14, 1, 9, 5, 13, 3, 11, 7, 15)

_GDN = lax.GatherDimensionNumbers(
    offset_dims=(), collapsed_slice_dims=(0,), start_index_map=(0,))


def _shuffle(x, idx):
    return lax.gather(x, idx[:, None], dimension_numbers=_GDN,
                      slice_sizes=(1,),
                      mode=lax.GatherScatterMode.PROMISE_IN_BOUNDS)


def _rsqrt_nr(x):
    i = lax.bitcast_convert_type(x, jnp.int32)
    y = lax.bitcast_convert_type(jnp.int32(0x5F3759DF) - (i >> 1),
                                 jnp.float32)
    for _ in range(3):
        y = y * (1.5 - 0.5 * x * y * y)
    return y


def _reduce16(vecs, lanes):
    for k in (8, 4, 2, 1):
        mask = (lanes & k) == 0
        perm = lanes ^ k
        vecs = [jnp.where(mask, a + _shuffle(a, perm),
                          b + _shuffle(b, perm))
                for a, b in zip(vecs[0::2], vecs[1::2])]
    return vecs[0]


def _tc_relayout(emb_t, V, D):
    half = _VB // 2

    def body(src_ref, dst_ref):
        x = src_ref[...]
        z = jnp.concatenate([x[:, :half], x[:, half:]], axis=0)
        r = lax.broadcasted_iota(jnp.int32, (2 * D, 2 * D), 0)
        c = lax.broadcasted_iota(jnp.int32, (2 * D, 2 * D), 1)
        eye = (r == c).astype(jnp.bfloat16)
        a = z.astype(jnp.bfloat16)
        r1 = z - a.astype(jnp.float32)
        b = r1.astype(jnp.bfloat16)
        c2 = (r1 - b.astype(jnp.float32)).astype(jnp.bfloat16)
        dn = (((0,), (0,)), ((), ()))
        acc = lax.dot_general(a, eye, dn, preferred_element_type=jnp.float32)
        acc += lax.dot_general(b, eye, dn, preferred_element_type=jnp.float32)
        acc += lax.dot_general(c2, eye, dn, preferred_element_type=jnp.float32)
        dst_ref[...] = acc

    grid = (V + _VB - 1) // _VB
    return pl.pallas_call(
        body,
        grid=(grid,),
        in_specs=[pl.BlockSpec((D, _VB), lambda i: (0, i))],
        out_specs=pl.BlockSpec((half, 2 * D), lambda i: (i, 0)),
        out_shape=jax.ShapeDtypeStruct((grid * half, 2 * D), jnp.float32),
    )(emb_t)


def _sc_body(n, n_per_w, n_chunks):
    def body(idx_hbm, table_hbm, out_hbm, idx1_v, idx2_v, vdx1_v, vdx2_v,
             r1a, r2a, r1b, r2b, oa, ob, sem_a, sem_b, sem_o):
        wid = lax.axis_index("s") * 2 + lax.axis_index("c")
        base_w = wid * n_per_w
        pltpu.sync_copy(idx_hbm.at[pl.ds(base_w, n_per_w)], idx1_v)
        pltpu.sync_copy(idx_hbm.at[pl.ds(n + base_w, n_per_w)], idx2_v)

        def packed_row(v):
            return ((v >> 13) << 12) + (v & 4095)

        def halve(i, carry):
            o = i * _LANES
            vdx1_v[pl.ds(o, _LANES)] = packed_row(idx1_v[pl.ds(o, _LANES)])
            vdx2_v[pl.ds(o, _LANES)] = packed_row(idx2_v[pl.ds(o, _LANES)])
            return carry

        lax.fori_loop(0, n_per_w // _LANES, halve, 0)

        def fire(k, r1, r2, sem):
            off = k * _C
            pltpu.async_copy(
                table_hbm.at[vdx1_v.at[pl.ds(off, _SUB)]], r1, sem)
            pltpu.async_copy(
                table_hbm.at[vdx2_v.at[pl.ds(off, _SUB)]], r2, sem)

        def drain_rows(r1, r2, sem):
            pltpu.make_async_copy(
                table_hbm.at[pl.ds(0, _SUB)], r1, sem).wait()
            pltpu.make_async_copy(
                table_hbm.at[pl.ds(0, _SUB)], r2, sem).wait()

        def drain_out(ov):
            pltpu.make_async_copy(
                out_hbm.at[pl.ds(0, _C)], ov, sem_o).wait()

        def compute(k, r1, r2, ov):
            def group(g, carry):
                lanes = lax.iota(jnp.int32, _LANES)
                p0 = g * _LANES
                h1v = ((idx1_v[pl.ds(k * _C + p0, _LANES)] >> 12) & 1) * _D
                h2v = ((idx2_v[pl.ds(k * _C + p0, _LANES)] >> 12) & 1) * _D
                s12s, s11s, s22s = [], [], []
                for j in _BITREV:
                    p = p0 + j
                    h1 = h1v[j]
                    h2 = h2v[j]
                    a = [r1[p, pl.ds(h1 + t * _LANES, _LANES)]
                         for t in range(_D // _LANES)]
                    b = [r2[p, pl.ds(h2 + t * _LANES, _LANES)]
                         for t in range(_D // _LANES)]
                    s12s.append((a[0] * b[0] + a[1] * b[1])
                                + (a[2] * b[2] + a[3] * b[3]))
                    s11s.append((a[0] * a[0] + a[1] * a[1])
                                + (a[2] * a[2] + a[3] * a[3]))
                    s22s.append((b[0] * b[0] + b[1] * b[1])
                                + (b[2] * b[2] + b[3] * b[3]))
                d12 = _reduce16(s12s, lanes)
                d11 = _reduce16(s11s, lanes)
                d22 = _reduce16(s22s, lanes)
                prod = jnp.maximum(d11, 1e-24) * jnp.maximum(d22, 1e-24)
                ov[pl.ds(p0, _LANES)] = d12 * _rsqrt_nr(prod)
                return carry

            lax.fori_loop(0, _C // _LANES, group, 0)
            pltpu.async_copy(ov, out_hbm.at[pl.ds(base_w + k * _C, _C)],
                             sem_o)

        fire(0, r1a, r2a, sem_a)
        fire(1, r1b, r2b, sem_b)

        def step(i, carry):
            k0 = 2 * i
            k1 = k0 + 1
            drain_rows(r1a, r2a, sem_a)

            @pl.when(i > 0)
            def _():
                drain_out(oa)

            compute(k0, r1a, r2a, oa)

            @pl.when(k0 + 2 < n_chunks)
            def _():
                fire(k0 + 2, r1a, r2a, sem_a)

            drain_rows(r1b, r2b, sem_b)

            @pl.when(i > 0)
            def _():
                drain_out(ob)

            compute(k1, r1b, r2b, ob)

            @pl.when(k1 + 2 < n_chunks)
            def _():
                fire(k1 + 2, r1b, r2b, sem_b)

            return carry

        lax.fori_loop(0, n_chunks // 2, step, 0)
        drain_out(oa)
        drain_out(ob)

    return body


@jax.jit
def kernel(inputs, embeddings):
    two, B, L = inputs.shape
    V, D = embeddings.shape
    N = B * L
    n_per_w = N // _NW
    n_chunks = n_per_w // _C

    idx_flat = inputs.reshape(2 * N)
    table2 = _tc_relayout(embeddings.T, V, D)

    mesh = plsc.VectorSubcoreMesh(core_axis_name="c", subcore_axis_name="s")
    run = pl.kernel(
        _sc_body(N, n_per_w, n_chunks),
        out_type=jax.ShapeDtypeStruct((N,), jnp.float32),
        mesh=mesh,
        compiler_params=pltpu.CompilerParams(use_tc_tiling_on_sc=True),
        scratch_types=[
            pltpu.VMEM((n_per_w,), jnp.int32),
            pltpu.VMEM((n_per_w,), jnp.int32),
            pltpu.VMEM((n_per_w,), jnp.int32),
            pltpu.VMEM((n_per_w,), jnp.int32),
            pltpu.VMEM((_C, 2 * _D), jnp.float32),
            pltpu.VMEM((_C, 2 * _D), jnp.float32),
            pltpu.VMEM((_C, 2 * _D), jnp.float32),
            pltpu.VMEM((_C, 2 * _D), jnp.float32),
            pltpu.VMEM((_C,), jnp.float32),
            pltpu.VMEM((_C,), jnp.float32),
            pltpu.SemaphoreType.DMA,
            pltpu.SemaphoreType.DMA,
            pltpu.SemaphoreType.DMA,
        ],
    )
    out = run(idx_flat, table2)
    return out.reshape(B, L)


# speedup vs baseline: 3.5372x; 1.1317x over previous
import functools

import jax
import jax.numpy as jnp
import numpy as np
from jax import lax
from jax.experimental import pallas as pl
from jax.experimental.pallas import tpu as pltpu
from jax.experimental.pallas import tpu_sc as plsc

_D = 64
_LANES = 16
_NW = 32
_C = 256
_SUB = 128
_VB = 8192
_BITREV = (0, 8, 4, 12, 2, 10, 6, 14, 1, 9, 5, 13, 3, 11, 7, 15)

_GDN = lax.GatherDimensionNumbers(
    offset_dims=(), collapsed_slice_dims=(0,), start_index_map=(0,))


def _shuffle(x, idx):
    return lax.gather(x, idx[:, None], dimension_numbers=_GDN,
                      slice_sizes=(1,),
                      mode=lax.GatherScatterMode.PROMISE_IN_BOUNDS)


def _rsqrt_nr(x):
    i = lax.bitcast_convert_type(x, jnp.int32)
    y = lax.bitcast_convert_type(jnp.int32(0x5F3759DF) - (i >> 1),
                                 jnp.float32)
    for _ in range(3):
        y = y * (1.5 - 0.5 * x * y * y)
    return y


def _reduce16(vecs, lanes):
    for k in (8, 4, 2, 1):
        mask = (lanes & k) == 0
        perm = lanes ^ k
        vecs = [jnp.where(mask, a + _shuffle(a, perm),
                          b + _shuffle(b, perm))
                for a, b in zip(vecs[0::2], vecs[1::2])]
    return vecs[0]


def _tc_relayout(emb_t, V, D):
    half = _VB // 2

    def body(src_ref, dst_ref):
        x = src_ref[...]
        z = jnp.concatenate([x[:, :half], x[:, half:]], axis=0)
        r = lax.broadcasted_iota(jnp.int32, (2 * D, 2 * D), 0)
        c = lax.broadcasted_iota(jnp.int32, (2 * D, 2 * D), 1)
        eye = (r == c).astype(jnp.bfloat16)
        a = z.astype(jnp.bfloat16)
        r1 = z - a.astype(jnp.float32)
        b = r1.astype(jnp.bfloat16)
        c2 = (r1 - b.astype(jnp.float32)).astype(jnp.bfloat16)
        dn = (((0,), (0,)), ((), ()))
        acc = lax.dot_general(a, eye, dn, preferred_element_type=jnp.float32)
        acc += lax.dot_general(b, eye, dn, preferred_element_type=jnp.float32)
        acc += lax.dot_general(c2, eye, dn, preferred_element_type=jnp.float32)
        dst_ref[...] = acc

    grid = (V + _VB - 1) // _VB
    return pl.pallas_call(
        body,
        grid=(grid,),
        in_specs=[pl.BlockSpec((D, _VB), lambda i: (0, i))],
        out_specs=pl.BlockSpec((half, 2 * D), lambda i: (i, 0)),
        out_shape=jax.ShapeDtypeStruct((grid * half, 2 * D), jnp.float32),
    )(emb_t)


def _sc_body(n, n_per_w, n_chunks):
    nsub = _C // _SUB

    def body(idx_hbm, table_hbm, out_hbm, vdx1_v, vdx2_v,
             r1a, r2a, r1b, r2b, oa, ob, sem_a, sem_b, sem_o):
        wid = lax.axis_index("s") * 2 + lax.axis_index("c")
        base_w = wid * n_per_w
        pltpu.sync_copy(idx_hbm.at[pl.ds(base_w, n_per_w)], vdx1_v)
        pltpu.sync_copy(idx_hbm.at[pl.ds(n + base_w, n_per_w)], vdx2_v)

        def flat_row(v):
            return ((v >> 13) << 13) + ((v & 4095) << 1) + ((v >> 12) & 1)

        def reindex(i, carry):
            o = i * _LANES
            vdx1_v[pl.ds(o, _LANES)] = flat_row(vdx1_v[pl.ds(o, _LANES)])
            vdx2_v[pl.ds(o, _LANES)] = flat_row(vdx2_v[pl.ds(o, _LANES)])
            return carry

        lax.fori_loop(0, n_per_w // _LANES, reindex, 0)

        def fire(k, r1, r2, sem):
            for j in range(nsub):
                off = k * _C + j * _SUB
                pltpu.async_copy(
                    table_hbm.at[vdx1_v.at[pl.ds(off, _SUB)]],
                    r1.at[pl.ds(j * _SUB, _SUB)], sem)
                pltpu.async_copy(
                    table_hbm.at[vdx2_v.at[pl.ds(off, _SUB)]],
                    r2.at[pl.ds(j * _SUB, _SUB)], sem)

        def drain_rows(r1, r2, sem):
            for j in range(nsub):
                pltpu.make_async_copy(
                    table_hbm.at[pl.ds(0, _SUB)],
                    r1.at[pl.ds(j * _SUB, _SUB)], sem).wait()
                pltpu.make_async_copy(
                    table_hbm.at[pl.ds(0, _SUB)],
                    r2.at[pl.ds(j * _SUB, _SUB)], sem).wait()

        def drain_out(ov):
            pltpu.make_async_copy(
                out_hbm.at[pl.ds(0, _C)], ov, sem_o).wait()

        def compute(k, r1, r2, ov):
            def group(g, carry):
                lanes = lax.iota(jnp.int32, _LANES)
                p0 = g * _LANES
                s12s, s11s, s22s = [], [], []
                for j in _BITREV:
                    p = p0 + j
                    a = [r1[p, pl.ds(t * _LANES, _LANES)]
                         for t in range(_D // _LANES)]
                    b = [r2[p, pl.ds(t * _LANES, _LANES)]
                         for t in range(_D // _LANES)]
                    s12s.append((a[0] * b[0] + a[1] * b[1])
                                + (a[2] * b[2] + a[3] * b[3]))
                    s11s.append((a[0] * a[0] + a[1] * a[1])
                                + (a[2] * a[2] + a[3] * a[3]))
                    s22s.append((b[0] * b[0] + b[1] * b[1])
                                + (b[2] * b[2] + b[3] * b[3]))
                d12 = _reduce16(s12s, lanes)
                d11 = _reduce16(s11s, lanes)
                d22 = _reduce16(s22s, lanes)
                prod = jnp.maximum(d11, 1e-24) * jnp.maximum(d22, 1e-24)
                ov[pl.ds(p0, _LANES)] = d12 * _rsqrt_nr(prod)
                return carry

            lax.fori_loop(0, _C // _LANES, group, 0)
            pltpu.async_copy(ov, out_hbm.at[pl.ds(base_w + k * _C, _C)],
                             sem_o)

        fire(0, r1a, r2a, sem_a)
        fire(1, r1b, r2b, sem_b)

        def step(i, carry):
            k0 = 2 * i
            k1 = k0 + 1
            drain_rows(r1a, r2a, sem_a)

            @pl.when(i > 0)
            def _():
                drain_out(oa)

            compute(k0, r1a, r2a, oa)

            @pl.when(k0 + 2 < n_chunks)
            def _():
                fire(k0 + 2, r1a, r2a, sem_a)

            drain_rows(r1b, r2b, sem_b)

            @pl.when(i > 0)
            def _():
                drain_out(ob)

            compute(k1, r1b, r2b, ob)

            @pl.when(k1 + 2 < n_chunks)
            def _():
                fire(k1 + 2, r1b, r2b, sem_b)

            return carry

        lax.fori_loop(0, n_chunks // 2, step, 0)
        drain_out(oa)
        drain_out(ob)

    return body


@jax.jit
def kernel(inputs, embeddings):
    two, B, L = inputs.shape
    V, D = embeddings.shape
    N = B * L
    n_per_w = N // _NW
    n_chunks = n_per_w // _C

    idx_flat = inputs.reshape(2 * N)
    table2 = _tc_relayout(embeddings.T, V, D)
    table_lin = table2.reshape(2 * table2.shape[0], D)

    mesh = plsc.VectorSubcoreMesh(core_axis_name="c", subcore_axis_name="s")
    run = pl.kernel(
        _sc_body(N, n_per_w, n_chunks),
        out_type=jax.ShapeDtypeStruct((N,), jnp.float32),
        mesh=mesh,
        compiler_params=pltpu.CompilerParams(use_tc_tiling_on_sc=False),
        scratch_types=[
            pltpu.VMEM((n_per_w,), jnp.int32),
            pltpu.VMEM((n_per_w,), jnp.int32),
            pltpu.VMEM((_C, _D), jnp.float32),
            pltpu.VMEM((_C, _D), jnp.float32),
            pltpu.VMEM((_C, _D), jnp.float32),
            pltpu.VMEM((_C, _D), jnp.float32),
            pltpu.VMEM((_C,), jnp.float32),
            pltpu.VMEM((_C,), jnp.float32),
            pltpu.SemaphoreType.DMA,
            pltpu.SemaphoreType.DMA,
            pltpu.SemaphoreType.DMA,
        ],
    )
    out = run(idx_flat, table_lin)
    return out.reshape(B, L)
